# baseline (speedup 1.0000x reference)
import functools

import jax
import jax.numpy as jnp
from jax import lax
from jax.experimental import pallas as pl
from jax.experimental.pallas import tpu as pltpu
from jax.experimental.pallas import tpu_sc as plsc

ATO = 100000
EMBED = 128
VQ_DIM = 256

BLK = 1024
TEXT_BLKS = 98
ADD_BLKS = 8
PROJ_BLKS = 16
TBL_BLKS = TEXT_BLKS + ADD_BLKS + PROJ_BLKS
TBL_ROWS = TBL_BLKS * BLK
SHIFT = TEXT_BLKS * BLK - ATO

NC, NS, L = 2, 16, 16
NW = NC * NS
NTOK = 1024 * 200
CHUNK = NTOK // NW
BATCH = 128
NB = CHUNK // BATCH


def _build_table_body(tok_ref, add_ref, cb_ref, w_ref, out_ref):
    g = pl.program_id(0)

    @pl.when(g < TEXT_BLKS)
    def _():
        out_ref[...] = tok_ref[...]

    @pl.when((g >= TEXT_BLKS) & (g < TEXT_BLKS + ADD_BLKS))
    def _():
        out_ref[...] = add_ref[...]

    @pl.when(g >= TEXT_BLKS + ADD_BLKS)
    def _():
        out_ref[...] = lax.dot_general(
            cb_ref[...], w_ref[...],
            dimension_numbers=(((1,), (1,)), ((), ())),
            preferred_element_type=jnp.float32,
        )


def _build_table(token_embedding, added_embedding, vqgan_codebook, vqgan_proj_W):
    return pl.pallas_call(
        _build_table_body,
        grid=(TBL_BLKS,),
        in_specs=[
            pl.BlockSpec((BLK, EMBED), lambda g: (jnp.minimum(g, TEXT_BLKS - 1), 0)),
            pl.BlockSpec((BLK, EMBED), lambda g: (jnp.clip(g - TEXT_BLKS, 0, ADD_BLKS - 1), 0)),
            pl.BlockSpec((BLK, VQ_DIM), lambda g: (jnp.clip(g - TEXT_BLKS - ADD_BLKS, 0, PROJ_BLKS - 1), 0)),
            pl.BlockSpec((EMBED, VQ_DIM), lambda g: (0, 0)),
        ],
        out_specs=pl.BlockSpec((BLK, EMBED), lambda g: (g, 0)),
        out_shape=jax.ShapeDtypeStruct((TBL_ROWS, EMBED), jnp.float32),
    )(token_embedding, added_embedding, vqgan_codebook, vqgan_proj_W)


def _remap_body(x_ref, out_ref):
    v = x_ref[...]
    out_ref[...] = jnp.where(v >= ATO, v + SHIFT, v)


def _remap_ids(x_flat):
    return pl.pallas_call(
        _remap_body,
        out_shape=jax.ShapeDtypeStruct(x_flat.shape, jnp.int32),
    )(x_flat)


NBUF = 5


@functools.cache
def _sc_gather_fn():
    mesh = plsc.VectorSubcoreMesh(
        core_axis_name="c", subcore_axis_name="s", num_cores=NC, num_subcores=NS)
    return functools.partial(
        pl.kernel,
        out_type=jax.ShapeDtypeStruct((NTOK, EMBED), jnp.float32),
        mesh=mesh,
        scratch_types=(
            [pltpu.VMEM((NB, BATCH), jnp.int32)]
            + [pltpu.VMEM((BATCH, EMBED), jnp.float32) for _ in range(NBUF)]
            + [pltpu.SemaphoreType.DMA for _ in range(2 * NBUF)]
        ),
    )(_sc_gather_body)


def _sc_gather_body(x_hbm, tbl_hbm, out_hbm, idx_v, *scratch):
    bufs = scratch[:NBUF]
    gsems = scratch[NBUF:2 * NBUF]
    osems = scratch[2 * NBUF:]

    wid = lax.axis_index("s") * NC + lax.axis_index("c")
    base = wid * CHUNK

    pltpu.sync_copy(x_hbm.at[wid], idx_v)

    def g_start(k, b):
        pltpu.make_async_copy(tbl_hbm.at[idx_v.at[k]], bufs[b], gsems[b]).start()

    def g_wait(k, b):
        pltpu.make_async_copy(tbl_hbm.at[idx_v.at[k]], bufs[b], gsems[b]).wait()

    def o_copy(k, b):
        return pltpu.make_async_copy(
            bufs[b], out_hbm.at[pl.ds(base + k * BATCH, BATCH)], osems[b])

    for b in range(NBUF):
        g_start(b, b)

    def loop_body(i, carry):
        for b in range(NBUF):
            k = NBUF * i + b
            g_wait(k, b)
            o_copy(k, b).start()
            nk = k + NBUF

            @pl.when(nk < NB)
            def _():
                o_copy(k, b).wait()
                g_start(nk, b)
        return carry

    lax.fori_loop(0, NB // NBUF, loop_body, 0)

    for b in range(NBUF):
        o_copy(NB - NBUF + b, b).wait()


def kernel(x, token_embedding, added_embedding, vqgan_codebook, vqgan_proj_W):
    tbl = _build_table(token_embedding, added_embedding, vqgan_codebook, vqgan_proj_W)
    x_r = _remap_ids(x.reshape(NTOK // BATCH, BATCH))
    out = _sc_gather_fn()(x_r.reshape(NW, NB, BATCH), tbl)
    return out.reshape(x.shape[0], x.shape[1], EMBED)


# speedup vs baseline: 1.1631x; 1.1631x over previous
import functools

import jax
import jax.numpy as jnp
from jax import lax
from jax.experimental import pallas as pl
from jax.experimental.pallas import tpu as pltpu
from jax.experimental.pallas import tpu_sc as plsc

ATO = 100000
EMBED = 128
VQ_DIM = 256

BLK = 2048
TEXT_BLKS = 49
ADD_BLKS = 4
PROJ_BLKS = 8
TBL_BLKS = TEXT_BLKS + ADD_BLKS + PROJ_BLKS
TBL_ROWS = TBL_BLKS * BLK
SHIFT = TEXT_BLKS * BLK - ATO

NC, NS, L = 2, 16, 16
NW = NC * NS
NTOK = 1024 * 200
CHUNK = NTOK // NW
BATCH = 128
NB = CHUNK // BATCH


def _build_table_body(tok_ref, add_ref, cb_ref, w_ref, out_ref):
    g = pl.program_id(0)

    @pl.when(g < TEXT_BLKS)
    def _():
        out_ref[...] = tok_ref[...]

    @pl.when((g >= TEXT_BLKS) & (g < TEXT_BLKS + ADD_BLKS))
    def _():
        out_ref[...] = add_ref[...]

    @pl.when(g >= TEXT_BLKS + ADD_BLKS)
    def _():
        out_ref[...] = lax.dot_general(
            cb_ref[...], w_ref[...],
            dimension_numbers=(((1,), (1,)), ((), ())),
            preferred_element_type=jnp.float32,
        )


def _build_table(token_embedding, added_embedding, vqgan_codebook, vqgan_proj_W):
    return pl.pallas_call(
        _build_table_body,
        grid=(TBL_BLKS,),
        in_specs=[
            pl.BlockSpec((BLK, EMBED), lambda g: (jnp.minimum(g, TEXT_BLKS - 1), 0)),
            pl.BlockSpec((BLK, EMBED), lambda g: (jnp.clip(g - TEXT_BLKS, 0, ADD_BLKS - 1), 0)),
            pl.BlockSpec((BLK, VQ_DIM), lambda g: (jnp.clip(g - TEXT_BLKS - ADD_BLKS, 0, PROJ_BLKS - 1), 0)),
            pl.BlockSpec((EMBED, VQ_DIM), lambda g: (0, 0)),
        ],
        out_specs=pl.BlockSpec((BLK, EMBED), lambda g: (g, 0)),
        out_shape=jax.ShapeDtypeStruct((TBL_ROWS, EMBED), jnp.float32),
    )(token_embedding, added_embedding, vqgan_codebook, vqgan_proj_W)


def _remap_body(x_ref, out_ref):
    v = x_ref[...]
    out_ref[...] = jnp.where(v >= ATO, v + SHIFT, v)


def _remap_ids(x_flat):
    return pl.pallas_call(
        _remap_body,
        out_shape=jax.ShapeDtypeStruct(x_flat.shape, jnp.int32),
    )(x_flat)


NBUF = 5


@functools.cache
def _sc_gather_fn():
    mesh = plsc.VectorSubcoreMesh(
        core_axis_name="c", subcore_axis_name="s", num_cores=NC, num_subcores=NS)
    return functools.partial(
        pl.kernel,
        out_type=jax.ShapeDtypeStruct((NTOK, EMBED), jnp.float32),
        mesh=mesh,
        scratch_types=(
            [pltpu.VMEM((NB, BATCH), jnp.int32)]
            + [pltpu.VMEM((BATCH, EMBED), jnp.float32) for _ in range(NBUF)]
            + [pltpu.SemaphoreType.DMA for _ in range(2 * NBUF)]
        ),
    )(_sc_gather_body)


def _sc_gather_body(x_hbm, tbl_hbm, out_hbm, idx_v, *scratch):
    bufs = scratch[:NBUF]
    gsems = scratch[NBUF:2 * NBUF]
    osems = scratch[2 * NBUF:]

    wid = lax.axis_index("s") * NC + lax.axis_index("c")
    base = wid * CHUNK

    pltpu.sync_copy(x_hbm.at[wid], idx_v)

    def g_start(k, b):
        pltpu.make_async_copy(tbl_hbm.at[idx_v.at[k]], bufs[b], gsems[b]).start()

    def g_wait(k, b):
        pltpu.make_async_copy(tbl_hbm.at[idx_v.at[k]], bufs[b], gsems[b]).wait()

    def o_copy(k, b):
        return pltpu.make_async_copy(
            bufs[b], out_hbm.at[pl.ds(base + k * BATCH, BATCH)], osems[b])

    for b in range(NBUF):
        g_start(b, b)

    def loop_body(i, carry):
        for b in range(NBUF):
            k = NBUF * i + b
            g_wait(k, b)
            o_copy(k, b).start()
            nk = k + NBUF

            @pl.when(nk < NB)
            def _():
                o_copy(k, b).wait()
                g_start(nk, b)
        return carry

    lax.fori_loop(0, NB // NBUF, loop_body, 0)

    for b in range(NBUF):
        o_copy(NB - NBUF + b, b).wait()


def kernel(x, token_embedding, added_embedding, vqgan_codebook, vqgan_proj_W):
    tbl = _build_table(token_embedding, added_embedding, vqgan_codebook, vqgan_proj_W)
    x_r = _remap_ids(x.reshape(NTOK // BATCH, BATCH))
    out = _sc_gather_fn()(x_r.reshape(NW, NB, BATCH), tbl)
    return out.reshape(x.shape[0], x.shape[1], EMBED)


# speedup vs baseline: 1.2961x; 1.1143x over previous
import functools

import jax
import jax.numpy as jnp
from jax import lax
from jax.experimental import pallas as pl
from jax.experimental.pallas import tpu as pltpu
from jax.experimental.pallas import tpu_sc as plsc

ATO = 100000
EMBED = 128
VQ_DIM = 256

BLK = 4096
TEXT_BLKS = 25
ADD_BLKS = 2
PROJ_BLKS = 4
TBL_BLKS = TEXT_BLKS + ADD_BLKS + PROJ_BLKS
TBL_ROWS = TBL_BLKS * BLK
SHIFT = TEXT_BLKS * BLK - ATO

NC, NS, L = 2, 16, 16
NW = NC * NS
NTOK = 1024 * 200
CHUNK = NTOK // NW
BATCH = 128
NB = CHUNK // BATCH


def _build_table_body(tok_ref, add_ref, cb_ref, w_ref, out_ref):
    g = pl.program_id(0)

    @pl.when(g < TEXT_BLKS)
    def _():
        out_ref[...] = tok_ref[...]

    @pl.when((g >= TEXT_BLKS) & (g < TEXT_BLKS + ADD_BLKS))
    def _():
        out_ref[...] = add_ref[...]

    @pl.when(g >= TEXT_BLKS + ADD_BLKS)
    def _():
        out_ref[...] = lax.dot_general(
            cb_ref[...], w_ref[...],
            dimension_numbers=(((1,), (1,)), ((), ())),
            preferred_element_type=jnp.float32,
        )


def _build_table(token_embedding, added_embedding, vqgan_codebook, vqgan_proj_W):
    return pl.pallas_call(
        _build_table_body,
        grid=(TBL_BLKS,),
        in_specs=[
            pl.BlockSpec((BLK, EMBED), lambda g: (jnp.minimum(g, TEXT_BLKS - 1), 0)),
            pl.BlockSpec((BLK, EMBED), lambda g: (jnp.clip(g - TEXT_BLKS, 0, ADD_BLKS - 1), 0)),
            pl.BlockSpec((BLK, VQ_DIM), lambda g: (jnp.clip(g - TEXT_BLKS - ADD_BLKS, 0, PROJ_BLKS - 1), 0)),
            pl.BlockSpec((EMBED, VQ_DIM), lambda g: (0, 0)),
        ],
        out_specs=pl.BlockSpec((BLK, EMBED), lambda g: (g, 0)),
        out_shape=jax.ShapeDtypeStruct((TBL_ROWS, EMBED), jnp.float32),
    )(token_embedding, added_embedding, vqgan_codebook, vqgan_proj_W)


def _remap_body(x_ref, out_ref):
    v = x_ref[...]
    out_ref[...] = jnp.where(v >= ATO, v + SHIFT, v)


def _remap_ids(x_flat):
    return pl.pallas_call(
        _remap_body,
        out_shape=jax.ShapeDtypeStruct(x_flat.shape, jnp.int32),
    )(x_flat)


NBUF = 5


@functools.cache
def _sc_gather_fn():
    mesh = plsc.VectorSubcoreMesh(
        core_axis_name="c", subcore_axis_name="s", num_cores=NC, num_subcores=NS)
    return functools.partial(
        pl.kernel,
        out_type=jax.ShapeDtypeStruct((NTOK, EMBED), jnp.float32),
        mesh=mesh,
        scratch_types=(
            [pltpu.VMEM((NB, BATCH), jnp.int32)]
            + [pltpu.VMEM((BATCH, EMBED), jnp.float32) for _ in range(NBUF)]
            + [pltpu.SemaphoreType.DMA for _ in range(2 * NBUF)]
        ),
    )(_sc_gather_body)


def _sc_gather_body(x_hbm, tbl_hbm, out_hbm, idx_v, *scratch):
    bufs = scratch[:NBUF]
    gsems = scratch[NBUF:2 * NBUF]
    osems = scratch[2 * NBUF:]

    wid = lax.axis_index("s") * NC + lax.axis_index("c")
    base = wid * CHUNK

    pltpu.sync_copy(x_hbm.at[wid], idx_v)

    def g_start(k, b):
        pltpu.make_async_copy(tbl_hbm.at[idx_v.at[k]], bufs[b], gsems[b]).start()

    def g_wait(k, b):
        pltpu.make_async_copy(tbl_hbm.at[idx_v.at[k]], bufs[b], gsems[b]).wait()

    def o_copy(k, b):
        return pltpu.make_async_copy(
            bufs[b], out_hbm.at[pl.ds(base + k * BATCH, BATCH)], osems[b])

    for b in range(NBUF):
        g_start(b, b)

    def loop_body(i, carry):
        for b in range(NBUF):
            k = NBUF * i + b
            g_wait(k, b)
            o_copy(k, b).start()
            nk = k + NBUF

            @pl.when(nk < NB)
            def _():
                o_copy(k, b).wait()
                g_start(nk, b)
        return carry

    lax.fori_loop(0, NB // NBUF, loop_body, 0)

    for b in range(NBUF):
        o_copy(NB - NBUF + b, b).wait()


def kernel(x, token_embedding, added_embedding, vqgan_codebook, vqgan_proj_W):
    tbl = _build_table(token_embedding, added_embedding, vqgan_codebook, vqgan_proj_W)
    x_r = _remap_ids(x.reshape(NTOK // BATCH, BATCH))
    out = _sc_gather_fn()(x_r.reshape(NW, NB, BATCH), tbl)
    return out.reshape(x.shape[0], x.shape[1], EMBED)


# speedup vs baseline: 1.3325x; 1.0281x over previous
import functools

import jax
import jax.numpy as jnp
from jax import lax
from jax.experimental import pallas as pl
from jax.experimental.pallas import tpu as pltpu
from jax.experimental.pallas import tpu_sc as plsc

ATO = 100000
EMBED = 128
VQ_DIM = 256

BLK = 8192
TEXT_BLKS = 13
ADD_BLKS = 1
PROJ_BLKS = 2
TBL_BLKS = TEXT_BLKS + ADD_BLKS + PROJ_BLKS
TBL_ROWS = TBL_BLKS * BLK
SHIFT = TEXT_BLKS * BLK - ATO

NC, NS, L = 2, 16, 16
NW = NC * NS
NTOK = 1024 * 200
CHUNK = NTOK // NW
BATCH = 128
NB = CHUNK // BATCH


def _build_table_body(tok_ref, add_ref, cb_ref, w_ref, out_ref):
    g = pl.program_id(0)

    @pl.when(g < TEXT_BLKS)
    def _():
        out_ref[...] = tok_ref[...]

    @pl.when((g >= TEXT_BLKS) & (g < TEXT_BLKS + ADD_BLKS))
    def _():
        out_ref[...] = add_ref[...]

    @pl.when(g >= TEXT_BLKS + ADD_BLKS)
    def _():
        out_ref[...] = lax.dot_general(
            cb_ref[...], w_ref[...],
            dimension_numbers=(((1,), (1,)), ((), ())),
            preferred_element_type=jnp.float32,
        )


def _build_table(token_embedding, added_embedding, vqgan_codebook, vqgan_proj_W):
    return pl.pallas_call(
        _build_table_body,
        grid=(TBL_BLKS,),
        in_specs=[
            pl.BlockSpec((BLK, EMBED), lambda g: (jnp.minimum(g, TEXT_BLKS - 1), 0)),
            pl.BlockSpec((BLK, EMBED), lambda g: (jnp.clip(g - TEXT_BLKS, 0, ADD_BLKS - 1), 0)),
            pl.BlockSpec((BLK, VQ_DIM), lambda g: (jnp.clip(g - TEXT_BLKS - ADD_BLKS, 0, PROJ_BLKS - 1), 0)),
            pl.BlockSpec((EMBED, VQ_DIM), lambda g: (0, 0)),
        ],
        out_specs=pl.BlockSpec((BLK, EMBED), lambda g: (g, 0)),
        out_shape=jax.ShapeDtypeStruct((TBL_ROWS, EMBED), jnp.float32),
    )(token_embedding, added_embedding, vqgan_codebook, vqgan_proj_W)


def _remap_body(x_ref, out_ref):
    v = x_ref[...]
    out_ref[...] = jnp.where(v >= ATO, v + SHIFT, v)


def _remap_ids(x_flat):
    return pl.pallas_call(
        _remap_body,
        out_shape=jax.ShapeDtypeStruct(x_flat.shape, jnp.int32),
    )(x_flat)


NBUF = 5


@functools.cache
def _sc_gather_fn():
    mesh = plsc.VectorSubcoreMesh(
        core_axis_name="c", subcore_axis_name="s", num_cores=NC, num_subcores=NS)
    return functools.partial(
        pl.kernel,
        out_type=jax.ShapeDtypeStruct((NTOK, EMBED), jnp.float32),
        mesh=mesh,
        scratch_types=(
            [pltpu.VMEM((NB, BATCH), jnp.int32)]
            + [pltpu.VMEM((BATCH, EMBED), jnp.float32) for _ in range(NBUF)]
            + [pltpu.SemaphoreType.DMA for _ in range(2 * NBUF)]
        ),
    )(_sc_gather_body)


def _sc_gather_body(x_hbm, tbl_hbm, out_hbm, idx_v, *scratch):
    bufs = scratch[:NBUF]
    gsems = scratch[NBUF:2 * NBUF]
    osems = scratch[2 * NBUF:]

    wid = lax.axis_index("s") * NC + lax.axis_index("c")
    base = wid * CHUNK

    pltpu.sync_copy(x_hbm.at[wid], idx_v)

    def g_start(k, b):
        pltpu.make_async_copy(tbl_hbm.at[idx_v.at[k]], bufs[b], gsems[b]).start()

    def g_wait(k, b):
        pltpu.make_async_copy(tbl_hbm.at[idx_v.at[k]], bufs[b], gsems[b]).wait()

    def o_copy(k, b):
        return pltpu.make_async_copy(
            bufs[b], out_hbm.at[pl.ds(base + k * BATCH, BATCH)], osems[b])

    for b in range(NBUF):
        g_start(b, b)

    def loop_body(i, carry):
        for b in range(NBUF):
            k = NBUF * i + b
            g_wait(k, b)
            o_copy(k, b).start()
            nk = k + NBUF

            @pl.when(nk < NB)
            def _():
                o_copy(k, b).wait()
                g_start(nk, b)
        return carry

    lax.fori_loop(0, NB // NBUF, loop_body, 0)

    for b in range(NBUF):
        o_copy(NB - NBUF + b, b).wait()


def kernel(x, token_embedding, added_embedding, vqgan_codebook, vqgan_proj_W):
    tbl = _build_table(token_embedding, added_embedding, vqgan_codebook, vqgan_proj_W)
    x_r = _remap_ids(x.reshape(NTOK // BATCH, BATCH))
    out = _sc_gather_fn()(x_r.reshape(NW, NB, BATCH), tbl)
    return out.reshape(x.shape[0], x.shape[1], EMBED)
